# Initial kernel scaffold; baseline (speedup 1.0000x reference)
import jax
import jax.numpy as jnp
from jax.experimental import pallas as pl


def kernel(z, weight):
    raise NotImplementedError("write your pallas kernel here")


# speedup vs baseline: 1.1030x; 1.1030x over previous
import functools

import jax
import jax.numpy as jnp
from jax.experimental import pallas as pl
from jax.experimental.pallas import tpu as pltpu

N_E = 8192
E_DIM = 256
BETA = 0.25
B_TOK = 4096

T_TILE = 512
K_TILE = 2048
T_GRID = B_TOK // T_TILE
K_GRID = N_E // K_TILE

F_TILE = 128
F_GRID = B_TOK // F_TILE


def _argmin_body(z_ref, w_ref, zsq_ref, wsq_ref, idx_ref, minv_ref):
    t = pl.program_id(0)
    k = pl.program_id(1)

    @pl.when(k == 0)
    def _():
        minv_ref[...] = jnp.full((T_TILE, 1), jnp.inf, jnp.float32)
        idx_ref[...] = jnp.zeros((T_TILE, 1), jnp.int32)

    z = z_ref[...]
    wt = w_ref[pl.ds(k * K_TILE, K_TILE), :]

    zsq = zsq_ref[...]
    wsq = wsq_ref[:, pl.ds(k * K_TILE, K_TILE)]
    s = jax.lax.dot_general(z, wt, (((1,), (1,)), ((), ())),
                            preferred_element_type=jnp.float32)
    d = (zsq + wsq) - 2.0 * s

    m = jnp.min(d, axis=1, keepdims=True)
    gidx = jax.lax.broadcasted_iota(jnp.int32, (T_TILE, K_TILE), 1) + k * K_TILE
    lidx = jnp.min(jnp.where(d == m, gidx, jnp.int32(2**31 - 1)),
                   axis=1, keepdims=True)
    better = m < minv_ref[...]
    idx_ref[...] = jnp.where(better, lidx, idx_ref[...])
    minv_ref[...] = jnp.where(better, m, minv_ref[...])


def _finish_body(idx_v_ref, idx_s_ref, zt_ref, w_ref,
                 enc_ref, out_ref, loss_ref, perp_ref,
                 zq_ref, hist_ref, acc_ref):
    t = pl.program_id(0)

    @pl.when(t == 0)
    def _():
        hist_ref[...] = jnp.zeros((1, N_E), jnp.float32)
        acc_ref[0] = 0.0

    idx_col = idx_v_ref[...]
    ii = jax.lax.broadcasted_iota(jnp.int32, (F_TILE, N_E), 1)
    onehot = jnp.where(ii == idx_col, 1.0, 0.0).astype(jnp.float32)
    enc_ref[...] = onehot
    hist_ref[...] += jnp.sum(onehot, axis=0, keepdims=True)

    def gather_one(i, _):
        row_idx = idx_s_ref[i, 0]
        zq_ref[pl.ds(i, 1), :] = w_ref[pl.ds(row_idx, 1), :]
        return 0
    jax.lax.fori_loop(0, F_TILE, gather_one, 0, unroll=4)

    zt = zt_ref[...]
    zq = zq_ref[...]
    diff = zq - zt
    out_ref[...] = zt + diff
    acc_ref[0] += jnp.sum(diff * diff)

    @pl.when(t == F_GRID - 1)
    def _():
        loss_ref[0, 0] = BETA * acc_ref[0] / (B_TOK * E_DIM)
        avg = hist_ref[...] / B_TOK
        ent = jnp.sum(avg * jnp.log(avg + 1e-10))
        perp_ref[0, 0] = jnp.exp(-ent)


@jax.jit
def kernel(z, weight):
    zt = jnp.transpose(z, (0, 2, 3, 4, 1))
    zf = zt.reshape(B_TOK, E_DIM).astype(jnp.float32)
    w = weight.astype(jnp.float32)

    zb = zf.astype(jnp.bfloat16)
    wb = w.astype(jnp.bfloat16)
    zsq = jnp.sum(zf ** 2, axis=1, keepdims=True)
    wsq = jnp.sum(w ** 2, axis=1).reshape(1, N_E)

    idx2 = pl.pallas_call(
        _argmin_body,
        grid=(T_GRID, K_GRID),
        in_specs=[
            pl.BlockSpec((T_TILE, E_DIM), lambda t, k: (t, 0)),
            pl.BlockSpec((N_E, E_DIM), lambda t, k: (0, 0)),
            pl.BlockSpec((T_TILE, 1), lambda t, k: (t, 0)),
            pl.BlockSpec((1, N_E), lambda t, k: (0, 0)),
        ],
        out_specs=pl.BlockSpec((T_TILE, 1), lambda t, k: (t, 0)),
        out_shape=jax.ShapeDtypeStruct((B_TOK, 1), jnp.int32),
        scratch_shapes=[pltpu.VMEM((T_TILE, 1), jnp.float32)],
    )(zb, wb, zsq, wsq)

    enc, out_flat, loss, perp = pl.pallas_call(
        _finish_body,
        grid=(F_GRID,),
        in_specs=[
            pl.BlockSpec((F_TILE, 1), lambda t: (t, 0)),
            pl.BlockSpec((F_TILE, 1), lambda t: (t, 0),
                         memory_space=pltpu.SMEM),
            pl.BlockSpec((F_TILE, E_DIM), lambda t: (t, 0)),
            pl.BlockSpec((N_E, E_DIM), lambda t: (0, 0)),
        ],
        out_specs=[
            pl.BlockSpec((F_TILE, N_E), lambda t: (t, 0)),
            pl.BlockSpec((F_TILE, E_DIM), lambda t: (t, 0)),
            pl.BlockSpec((1, 1), lambda t: (0, 0), memory_space=pltpu.SMEM),
            pl.BlockSpec((1, 1), lambda t: (0, 0), memory_space=pltpu.SMEM),
        ],
        out_shape=[
            jax.ShapeDtypeStruct((B_TOK, N_E), jnp.float32),
            jax.ShapeDtypeStruct((B_TOK, E_DIM), jnp.float32),
            jax.ShapeDtypeStruct((1, 1), jnp.float32),
            jax.ShapeDtypeStruct((1, 1), jnp.float32),
        ],
        scratch_shapes=[
            pltpu.VMEM((F_TILE, E_DIM), jnp.float32),
            pltpu.VMEM((1, N_E), jnp.float32),
            pltpu.SMEM((1,), jnp.float32),
        ],
    )(idx2, idx2, zf, w)

    out = jnp.transpose(out_flat.reshape(zt.shape), (0, 4, 1, 2, 3))
    return (out, loss.reshape(()), perp.reshape(()), enc,
            idx2.reshape(B_TOK))
